# baseline (speedup 1.0000x reference)
import functools

import jax
import jax.numpy as jnp
import numpy as np
from jax.experimental import pallas as pl
from jax.experimental.pallas import tpu as pltpu

_BLOCK = 32
_NLOCAL = 2
_STRIDE = 8
_TQ = 256
_SUPER = _STRIDE * _BLOCK
_LOCW = _TQ + 2 * _BLOCK
_NEG = -1e30


def _local_start(t, S):
    return min(max(t * _TQ - _BLOCK, 0), S - _LOCW)


def _make_biases(T, S):
    ns = (S // _SUPER) * _BLOCK
    rows = np.arange(T)[None, :] // _BLOCK
    cs = np.arange(ns)[:, None] // _BLOCK * _STRIDE
    bias_s = np.where(np.abs(rows - cs) >= _NLOCAL, 0.0, _NEG).astype(np.float32)

    bias_l = np.full((_LOCW, T), _NEG, dtype=np.float32)
    for t in range(T // _TQ):
        start = _local_start(t, S)
        r = np.arange(t * _TQ, (t + 1) * _TQ)[None, :] // _BLOCK
        c = start // _BLOCK + np.arange(_LOCW)[:, None] // _BLOCK
        bias_l[:, t * _TQ:(t + 1) * _TQ] = np.where(
            np.abs(r - c) < _NLOCAL, 0.0, _NEG)
    return bias_s, bias_l


def _attn_kernel(H, E, HG, q_ref, k_ref, v_ref, bs_ref, bl_ref, o_ref,
                 ks_ref, vs_ref):
    S = k_ref.shape[2]
    n_super = S // _SUPER
    temp = 1.0 / float(np.sqrt(E))

    dk = (((0,), (0,)), ((), ()))
    dv = (((1,), (0,)), ((), ()))
    for hh in range(HG):
        for i in range(n_super):
            ks_ref[:, i * _BLOCK:(i + 1) * _BLOCK] = \
                k_ref[hh, :, i * _SUPER:i * _SUPER + _BLOCK]
            vs_ref[:, i * _BLOCK:(i + 1) * _BLOCK] = \
                v_ref[hh, :, i * _SUPER:i * _SUPER + _BLOCK]
        ks = ks_ref[...]
        vs = vs_ref[...]

        for t in range(q_ref.shape[2] // _TQ):
            start = _local_start(t, S)
            c0 = t * _TQ
            q = q_ref[hh, :, c0:c0 + _TQ] * temp
            kl = k_ref[hh, :, start:start + _LOCW]
            vl = v_ref[hh, :, start:start + _LOCW]
            bs = bs_ref[:, c0:c0 + _TQ]
            bl = bl_ref[:, c0:c0 + _TQ]

            ss = jax.lax.dot_general(ks, q, dk,
                                     preferred_element_type=jnp.float32) + bs
            sl = jax.lax.dot_general(kl, q, dk,
                                     preferred_element_type=jnp.float32) + bl

            m = jnp.maximum(jnp.max(ss, axis=0), jnp.max(sl, axis=0))
            ps = jnp.exp(ss - m[None, :])
            plc = jnp.exp(sl - m[None, :])
            denom = jnp.sum(ps, axis=0) + jnp.sum(plc, axis=0)

            out = jax.lax.dot_general(vs, ps, dv,
                                      preferred_element_type=jnp.float32)
            out = out + jax.lax.dot_general(vl, plc, dv,
                                            preferred_element_type=jnp.float32)
            o_ref[hh, :, c0:c0 + _TQ] = out / denom[None, :]


def kernel(query, key, value):
    B, T, H, E = query.shape
    S = key.shape[1]
    qt = jnp.transpose(query[0], (1, 2, 0))
    kt = jnp.transpose(key[0], (1, 2, 0))
    vt = jnp.transpose(value[0], (1, 2, 0))
    ns = (S // _SUPER) * _BLOCK
    bias_s, bias_l = _make_biases(T, S)

    HG = 4
    out = pl.pallas_call(
        functools.partial(_attn_kernel, H, E, HG),
        grid=(H // HG,),
        in_specs=[
            pl.BlockSpec((HG, E, T), lambda h: (h, 0, 0)),
            pl.BlockSpec((HG, E, S), lambda h: (h, 0, 0)),
            pl.BlockSpec((HG, E, S), lambda h: (h, 0, 0)),
            pl.BlockSpec((ns, T), lambda h: (0, 0)),
            pl.BlockSpec((_LOCW, T), lambda h: (0, 0)),
        ],
        out_specs=pl.BlockSpec((HG, E, T), lambda h: (h, 0, 0)),
        out_shape=jax.ShapeDtypeStruct((H, E, T), jnp.float32),
        scratch_shapes=[
            pltpu.VMEM((E, ns), jnp.float32),
            pltpu.VMEM((E, ns), jnp.float32),
        ],
    )(qt, kt, vt, jnp.asarray(bias_s), jnp.asarray(bias_l))
    return jnp.transpose(out, (2, 0, 1))[None]


# speedup vs baseline: 1.0024x; 1.0024x over previous
import functools

import jax
import jax.numpy as jnp
import numpy as np
from jax.experimental import pallas as pl
from jax.experimental.pallas import tpu as pltpu

_BLOCK = 32
_NLOCAL = 2
_STRIDE = 8
_TQ = 256
_SUPER = _STRIDE * _BLOCK
_LOCW = _TQ + 2 * _BLOCK
_NEG = -1e30


def _local_start(t, S):
    return min(max(t * _TQ - _BLOCK, 0), S - _LOCW)


def _make_biases(T, S):
    ns = (S // _SUPER) * _BLOCK
    rows = np.arange(T)[None, :] // _BLOCK
    cs = np.arange(ns)[:, None] // _BLOCK * _STRIDE
    bias_s = np.where(np.abs(rows - cs) >= _NLOCAL, 0.0, _NEG).astype(np.float32)

    bias_l = np.full((_LOCW, T), _NEG, dtype=np.float32)
    for t in range(T // _TQ):
        start = _local_start(t, S)
        r = np.arange(t * _TQ, (t + 1) * _TQ)[None, :] // _BLOCK
        c = start // _BLOCK + np.arange(_LOCW)[:, None] // _BLOCK
        bias_l[:, t * _TQ:(t + 1) * _TQ] = np.where(
            np.abs(r - c) < _NLOCAL, 0.0, _NEG)
    return bias_s, bias_l


def _attn_kernel(H, E, HG, q_ref, k_ref, v_ref, bs_ref, bl_ref, o_ref,
                 ks_ref, vs_ref):
    S = k_ref.shape[2]
    n_super = S // _SUPER
    temp = 1.0 / float(np.sqrt(E))

    dk = (((0,), (0,)), ((), ()))
    dv = (((1,), (0,)), ((), ()))
    for hh in range(HG):
        for i in range(n_super):
            ks_ref[:, i * _BLOCK:(i + 1) * _BLOCK] = \
                k_ref[hh, :, i * _SUPER:i * _SUPER + _BLOCK]
            vs_ref[:, i * _BLOCK:(i + 1) * _BLOCK] = \
                v_ref[hh, :, i * _SUPER:i * _SUPER + _BLOCK]
        ks = ks_ref[...]
        vs = vs_ref[...]

        for t in range(q_ref.shape[2] // _TQ):
            start = _local_start(t, S)
            c0 = t * _TQ
            q = q_ref[hh, :, c0:c0 + _TQ] * temp
            kl = k_ref[hh, :, start:start + _LOCW]
            vl = v_ref[hh, :, start:start + _LOCW]
            bs = bs_ref[:, c0:c0 + _TQ]
            bl = bl_ref[:, c0:c0 + _TQ]

            ss = jax.lax.dot_general(ks, q, dk,
                                     preferred_element_type=jnp.float32) + bs
            sl = jax.lax.dot_general(kl, q, dk,
                                     preferred_element_type=jnp.float32) + bl

            m = jnp.maximum(jnp.max(ss, axis=0), jnp.max(sl, axis=0))
            ps = jnp.exp(ss - m[None, :])
            plc = jnp.exp(sl - m[None, :])
            recip = 1.0 / (jnp.sum(ps, axis=0) + jnp.sum(plc, axis=0))

            out = jax.lax.dot_general(vs, ps, dv,
                                      preferred_element_type=jnp.float32)
            out = out + jax.lax.dot_general(vl, plc, dv,
                                            preferred_element_type=jnp.float32)
            o_ref[hh, :, c0:c0 + _TQ] = out * recip[None, :]


def kernel(query, key, value):
    B, T, H, E = query.shape
    S = key.shape[1]
    qt = jnp.transpose(query[0], (1, 2, 0))
    kt = jnp.transpose(key[0], (1, 2, 0))
    vt = jnp.transpose(value[0], (1, 2, 0))
    ns = (S // _SUPER) * _BLOCK
    bias_s, bias_l = _make_biases(T, S)

    HG = 2
    out = pl.pallas_call(
        functools.partial(_attn_kernel, H, E, HG),
        grid=(H // HG,),
        in_specs=[
            pl.BlockSpec((HG, E, T), lambda h: (h, 0, 0)),
            pl.BlockSpec((HG, E, S), lambda h: (h, 0, 0)),
            pl.BlockSpec((HG, E, S), lambda h: (h, 0, 0)),
            pl.BlockSpec((ns, T), lambda h: (0, 0)),
            pl.BlockSpec((_LOCW, T), lambda h: (0, 0)),
        ],
        out_specs=pl.BlockSpec((HG, E, T), lambda h: (h, 0, 0)),
        out_shape=jax.ShapeDtypeStruct((H, E, T), jnp.float32),
        scratch_shapes=[
            pltpu.VMEM((E, ns), jnp.float32),
            pltpu.VMEM((E, ns), jnp.float32),
        ],
    )(qt, kt, vt, jnp.asarray(bias_s), jnp.asarray(bias_l))
    return jnp.transpose(out, (2, 0, 1))[None]


# speedup vs baseline: 1.0180x; 1.0156x over previous
import functools

import jax
import jax.numpy as jnp
import numpy as np
from jax.experimental import pallas as pl
from jax.experimental.pallas import tpu as pltpu

_BLOCK = 32
_NLOCAL = 2
_STRIDE = 8
_TQ = 256
_SUPER = _STRIDE * _BLOCK
_LOCW = _TQ + 2 * _BLOCK
_NEG = -1e30


def _local_start(t, S):
    return min(max(t * _TQ - _BLOCK, 0), S - _LOCW)


def _make_biases(T, S):
    ns = (S // _SUPER) * _BLOCK
    rows = np.arange(T)[None, :] // _BLOCK
    cs = np.arange(ns)[:, None] // _BLOCK * _STRIDE
    bias_s = np.where(np.abs(rows - cs) >= _NLOCAL, 0.0, _NEG).astype(np.float32)

    n_t = T // _TQ
    bias_l = np.full((_LOCW, 3 * _TQ), _NEG, dtype=np.float32)
    for p, t in enumerate((0, 1, n_t - 1)):
        start = _local_start(t, S)
        r = np.arange(t * _TQ, (t + 1) * _TQ)[None, :] // _BLOCK
        c = start // _BLOCK + np.arange(_LOCW)[:, None] // _BLOCK
        bias_l[:, p * _TQ:(p + 1) * _TQ] = np.where(
            np.abs(r - c) < _NLOCAL, 0.0, _NEG)
    return bias_s, bias_l


def _attn_kernel(H, E, HG, q_ref, k_ref, v_ref, bs_ref, bl_ref, o_ref,
                 ks_ref, vs_ref):
    S = k_ref.shape[2]
    n_super = S // _SUPER
    temp = 1.0 / float(np.sqrt(E))

    dk = (((0,), (0,)), ((), ()))
    dv = (((1,), (0,)), ((), ()))
    for hh in range(HG):
        for i in range(n_super):
            ks_ref[:, i * _BLOCK:(i + 1) * _BLOCK] = \
                k_ref[hh, :, i * _SUPER:i * _SUPER + _BLOCK]
            vs_ref[:, i * _BLOCK:(i + 1) * _BLOCK] = \
                v_ref[hh, :, i * _SUPER:i * _SUPER + _BLOCK]
        ks = ks_ref[...]
        vs = vs_ref[...]

        for t in range(q_ref.shape[2] // _TQ):
            start = _local_start(t, S)
            c0 = t * _TQ
            q = q_ref[hh, :, c0:c0 + _TQ] * temp
            kl = k_ref[hh, :, start:start + _LOCW]
            vl = v_ref[hh, :, start:start + _LOCW]
            n_t = q_ref.shape[2] // _TQ
            p = 0 if t == 0 else (2 if t == n_t - 1 else 1)
            bs = bs_ref[:, c0:c0 + _TQ]
            bl = bl_ref[:, p * _TQ:(p + 1) * _TQ]

            ss = jax.lax.dot_general(ks, q, dk,
                                     preferred_element_type=jnp.float32) + bs
            sl = jax.lax.dot_general(kl, q, dk,
                                     preferred_element_type=jnp.float32) + bl

            m = jnp.maximum(jnp.max(ss, axis=0), jnp.max(sl, axis=0))
            ps = jnp.exp(ss - m[None, :])
            plc = jnp.exp(sl - m[None, :])
            recip = 1.0 / (jnp.sum(ps, axis=0) + jnp.sum(plc, axis=0))

            out = jax.lax.dot_general(vs, ps, dv,
                                      preferred_element_type=jnp.float32)
            out = out + jax.lax.dot_general(vl, plc, dv,
                                            preferred_element_type=jnp.float32)
            o_ref[hh, :, c0:c0 + _TQ] = out * recip[None, :]


def kernel(query, key, value):
    B, T, H, E = query.shape
    S = key.shape[1]
    qt = jnp.transpose(query[0], (1, 2, 0))
    kt = jnp.transpose(key[0], (1, 2, 0))
    vt = jnp.transpose(value[0], (1, 2, 0))
    ns = (S // _SUPER) * _BLOCK
    bias_s, bias_l = _make_biases(T, S)

    HG = 2
    out = pl.pallas_call(
        functools.partial(_attn_kernel, H, E, HG),
        grid=(H // HG,),
        in_specs=[
            pl.BlockSpec((HG, E, T), lambda h: (h, 0, 0)),
            pl.BlockSpec((HG, E, S), lambda h: (h, 0, 0)),
            pl.BlockSpec((HG, E, S), lambda h: (h, 0, 0)),
            pl.BlockSpec((ns, T), lambda h: (0, 0)),
            pl.BlockSpec((_LOCW, 3 * _TQ), lambda h: (0, 0)),
        ],
        out_specs=pl.BlockSpec((HG, E, T), lambda h: (h, 0, 0)),
        out_shape=jax.ShapeDtypeStruct((H, E, T), jnp.float32),
        scratch_shapes=[
            pltpu.VMEM((E, ns), jnp.float32),
            pltpu.VMEM((E, ns), jnp.float32),
        ],
    )(qt, kt, vt, jnp.asarray(bias_s), jnp.asarray(bias_l))
    return jnp.transpose(out, (2, 0, 1))[None]
